# baseline (speedup 1.0000x reference)
import functools

import jax
import jax.numpy as jnp
from jax import lax
from jax.experimental import pallas as pl
from jax.experimental.pallas import tpu as pltpu
from jax.experimental.pallas import tpu_sc as plsc

N = 10000
E = 320000
D_IN = 128
D_HID = 150
D_OUT = 128
DD = 8

NC, NS = 2, 16
NW = NC * NS
EW = E // NW
K = 96
NCH = 108
EWP = NCH * K
RPS = N // NS
NB = 4

_MESH = plsc.VectorSubcoreMesh(
    core_axis_name="c", subcore_axis_name="s", num_cores=NC, num_subcores=NS)


def _make_agg(with_deg, dtype=jnp.float32):
  out_type = [jax.ShapeDtypeStruct((NC, N, D_IN), dtype)]
  scratch = [
      pltpu.VMEM((NCH, K), jnp.int32),
      pltpu.VMEM((NCH, K), jnp.int32),
      [pltpu.VMEM((K, D_IN), dtype)] * NB,
      pltpu.VMEM_SHARED((N + 16, D_IN), dtype),
      [pltpu.SemaphoreType.DMA] * NB,
      [pltpu.SemaphoreType.DMA] * NB,
  ]
  if with_deg:
    out_type.append(jax.ShapeDtypeStruct((NC, N, DD), jnp.float32))
    scratch += [
        pltpu.VMEM((K, DD), jnp.float32),
        pltpu.VMEM_SHARED((N + 16, DD), jnp.float32),
        [pltpu.SemaphoreType.DMA] * NB,
    ]

  @functools.partial(
      pl.kernel,
      mesh=_MESH,
      compiler_params=pltpu.CompilerParams(use_tc_tiling_on_sc=False),
      out_type=tuple(out_type),
      scratch_types=scratch,
  )
  def agg(*refs):
    if with_deg:
      (feat_hbm, src_hbm, dst_hbm, zeros_hbm, zeros8_hbm, ones_hbm,
       out_hbm, outd_hbm, src_v, dst_v, bufs, acc, gsems, ssems,
       ones_v, accd, dsems) = refs
    else:
      (feat_hbm, src_hbm, dst_hbm, zeros_hbm,
       out_hbm, src_v, dst_v, bufs, acc, gsems, ssems) = refs
    c = lax.axis_index("c")
    s = lax.axis_index("s")
    wid = s * NC + c
    base = s * RPS
    pltpu.sync_copy(zeros_hbm.at[pl.ds(base, RPS)], acc.at[pl.ds(base, RPS)])
    pltpu.sync_copy(src_hbm.at[wid], src_v)
    pltpu.sync_copy(dst_hbm.at[wid], dst_v)
    if with_deg:
      pltpu.sync_copy(zeros8_hbm.at[pl.ds(base, RPS)],
                      accd.at[pl.ds(base, RPS)])
      pltpu.sync_copy(ones_hbm, ones_v)
    plsc.subcore_barrier()

    def gath(j, b):
      pltpu.async_copy(feat_hbm.at[src_v.at[j]], bufs[b], gsems[b])

    def gwait(j, b):
      pltpu.make_async_copy(feat_hbm.at[src_v.at[j]], bufs[b], gsems[b]).wait()

    def scat(j, b):
      pltpu.async_copy(bufs[b], acc.at[dst_v.at[j]], ssems[b], add=True)
      if with_deg:
        pltpu.async_copy(ones_v, accd.at[dst_v.at[j]], dsems[b], add=True)

    def swait(j, b):
      pltpu.make_async_copy(bufs[b], acc.at[dst_v.at[j]], ssems[b]).wait()
      if with_deg:
        pltpu.make_async_copy(ones_v, accd.at[dst_v.at[j]], dsems[b]).wait()

    gath(0, 0)
    gath(1, 1)

    def body(i, carry):
      j0 = NB * i
      for t in range(NB):
        j = j0 + t
        b2 = (t + 2) % NB
        gwait(j, t)
        scat(j, t)
        if t < 2:
          @pl.when(i > 0)
          def _():
            swait(j - 2, b2)
        else:
          swait(j - 2, b2)

        @pl.when(j + 2 < NCH)
        def _():
          gath(j + 2, b2)

      return carry

    lax.fori_loop(0, NCH // NB, body, 0)
    swait(NCH - 2, 2)
    swait(NCH - 1, 3)
    plsc.subcore_barrier()
    pltpu.sync_copy(acc.at[pl.ds(base, RPS)], out_hbm.at[c, pl.ds(base, RPS)])
    if with_deg:
      pltpu.sync_copy(accd.at[pl.ds(base, RPS)],
                      outd_hbm.at[c, pl.ds(base, RPS)])

  return agg


_AGG1 = _make_agg(True, jnp.bfloat16)
_AGG2 = _make_agg(False, jnp.bfloat16)

_R = 1000


def _tc1(x, parts1, degp, w1s, w1n, b1, w2s, w2n, b2):
  def body(x_ref, p_ref, d_ref, w1s_ref, w1n_ref, b1_ref, w2s_ref, w2n_ref,
           b2_ref, p2_ref, s2_ref):
    agg = (p_ref[0].astype(jnp.float32)
           + p_ref[1].astype(jnp.float32))
    deg = d_ref[0, :, :1] + d_ref[1, :, :1]
    hn = agg * (1.0 / jnp.maximum(deg, 1.0))
    h1 = jnp.maximum(
        jnp.dot(x_ref[...], w1s_ref[...], preferred_element_type=jnp.float32)
        + jnp.dot(hn, w1n_ref[...], preferred_element_type=jnp.float32)
        + b1_ref[...], 0.0)
    p2_ref[...] = jnp.dot(
        h1, w2n_ref[...],
        preferred_element_type=jnp.float32).astype(jnp.bfloat16)
    s2_ref[...] = (jnp.dot(h1, w2s_ref[...], preferred_element_type=jnp.float32)
                   + b2_ref[...])

  return pl.pallas_call(
      body,
      grid=(N // _R,),
      in_specs=[
          pl.BlockSpec((_R, D_IN), lambda i: (i, 0)),
          pl.BlockSpec((NC, _R, D_IN), lambda i: (0, i, 0)),
          pl.BlockSpec((NC, _R, DD), lambda i: (0, i, 0)),
          pl.BlockSpec((D_IN, D_HID), lambda i: (0, 0)),
          pl.BlockSpec((D_IN, D_HID), lambda i: (0, 0)),
          pl.BlockSpec((1, D_HID), lambda i: (0, 0)),
          pl.BlockSpec((D_HID, D_OUT), lambda i: (0, 0)),
          pl.BlockSpec((D_HID, D_OUT), lambda i: (0, 0)),
          pl.BlockSpec((1, D_OUT), lambda i: (0, 0)),
      ],
      out_specs=[
          pl.BlockSpec((_R, D_OUT), lambda i: (i, 0)),
          pl.BlockSpec((_R, D_OUT), lambda i: (i, 0)),
      ],
      out_shape=[
          jax.ShapeDtypeStruct((N, D_OUT), jnp.bfloat16),
          jax.ShapeDtypeStruct((N, D_OUT), jnp.float32),
      ],
  )(x, parts1, degp, w1s, w1n, b1, w2s, w2n, b2)


def _tc2(s2, parts2, degp):
  def body(s2_ref, p_ref, d_ref, o_ref):
    deg = d_ref[0, :, :1] + d_ref[1, :, :1]
    agg = p_ref[0].astype(jnp.float32) + p_ref[1].astype(jnp.float32)
    o_ref[...] = s2_ref[...] + agg * (1.0 / jnp.maximum(deg, 1.0))

  return pl.pallas_call(
      body,
      grid=(N // _R,),
      in_specs=[
          pl.BlockSpec((_R, D_OUT), lambda i: (i, 0)),
          pl.BlockSpec((NC, _R, D_OUT), lambda i: (0, i, 0)),
          pl.BlockSpec((NC, _R, DD), lambda i: (0, i, 0)),
      ],
      out_specs=pl.BlockSpec((_R, D_OUT), lambda i: (i, 0)),
      out_shape=jax.ShapeDtypeStruct((N, D_OUT), jnp.float32),
  )(s2, parts2, degp)


def kernel(in_feat, edge_index, W1_self, W1_neigh, b1, W2_self, W2_neigh, b2):
  pad = ((0, 0), (0, EWP - EW))
  src = jnp.pad(edge_index[0].astype(jnp.int32).reshape(NW, EW), pad,
                constant_values=0).reshape(NW, NCH, K)
  dst = jnp.pad(edge_index[1].astype(jnp.int32).reshape(NW, EW), pad,
                constant_values=N).reshape(NW, NCH, K)
  zeros128 = jnp.zeros((N, D_IN), jnp.bfloat16)
  zeros8 = jnp.zeros((N, DD), jnp.float32)
  ones = jnp.ones((K, DD), jnp.float32)
  parts1, degp = _AGG1(in_feat.astype(jnp.bfloat16), src, dst, zeros128,
                       zeros8, ones)
  p2, s2 = _tc1(in_feat, parts1, degp, W1_self.T, W1_neigh.T,
                b1.reshape(1, -1), W2_self.T, W2_neigh.T, b2.reshape(1, -1))
  (parts2,) = _AGG2(p2, src, dst, zeros128)
  return _tc2(s2, parts2, degp)


# speedup vs baseline: 1.8501x; 1.8501x over previous
import functools

import jax
import jax.numpy as jnp
from jax import lax
from jax.experimental import pallas as pl
from jax.experimental.pallas import tpu as pltpu
from jax.experimental.pallas import tpu_sc as plsc

N = 10000
E = 320000
D_IN = 128
D_HID = 150
D_OUT = 128
DD = 8

NC, NS = 2, 16
NW = NC * NS
EW = E // NW
K = 96
NCH = 105
EWP = NCH * K
RPS = N // NS
NB = 2

_MESH = plsc.VectorSubcoreMesh(
    core_axis_name="c", subcore_axis_name="s", num_cores=NC, num_subcores=NS)


def _make_agg(with_deg, dtype=jnp.float32):
  out_type = [jax.ShapeDtypeStruct((NC, N, D_IN), dtype)]
  scratch = [
      pltpu.VMEM((NCH, K), jnp.int32),
      pltpu.VMEM((NCH, K), jnp.int32),
      [pltpu.VMEM((K, D_IN), dtype)] * NB,
      pltpu.VMEM_SHARED((N + 16, D_IN), dtype),
      [pltpu.SemaphoreType.DMA] * NB,
  ]
  if with_deg:
    out_type.append(jax.ShapeDtypeStruct((NC, N, DD), jnp.float32))
    scratch += [
        pltpu.VMEM((K, DD), jnp.float32),
        pltpu.VMEM_SHARED((N + 16, DD), jnp.float32),
    ]

  @functools.partial(
      pl.kernel,
      mesh=_MESH,
      compiler_params=pltpu.CompilerParams(use_tc_tiling_on_sc=False),
      out_type=tuple(out_type),
      scratch_types=scratch,
  )
  def agg(*refs):
    if with_deg:
      (feat_hbm, src_hbm, dst_hbm, zeros_hbm, zeros8_hbm, ones_hbm,
       out_hbm, outd_hbm, src_v, dst_v, bufs, acc, gsems,
       ones_v, accd) = refs
    else:
      (feat_hbm, src_hbm, dst_hbm, zeros_hbm,
       out_hbm, src_v, dst_v, bufs, acc, gsems) = refs
    c = lax.axis_index("c")
    s = lax.axis_index("s")
    wid = s * NC + c
    base = s * RPS
    pltpu.sync_copy(zeros_hbm.at[pl.ds(base, RPS)], acc.at[pl.ds(base, RPS)])
    pltpu.sync_copy(src_hbm.at[wid], src_v)
    pltpu.sync_copy(dst_hbm.at[wid], dst_v)
    if with_deg:
      pltpu.sync_copy(zeros8_hbm.at[pl.ds(base, RPS)],
                      accd.at[pl.ds(base, RPS)])
      pltpu.sync_copy(ones_hbm, ones_v)
    plsc.subcore_barrier()

    def gath(j, b):
      pltpu.async_copy(feat_hbm.at[src_v.at[j]], bufs[b], gsems[b])

    def scat(j, b):
      pltpu.make_async_copy(feat_hbm.at[src_v.at[j]], bufs[b], gsems[b]).wait()
      pltpu.sync_copy(bufs[b], acc.at[dst_v.at[j]], add=True)
      if with_deg:
        pltpu.sync_copy(ones_v, accd.at[dst_v.at[j]], add=True)

    gath(0, 0)

    def body(i, carry):
      j0 = 2 * i
      j1 = j0 + 1

      @pl.when(j1 < NCH)
      def _():
        gath(j1, 1)

      scat(j0, 0)

      @pl.when(j0 + 2 < NCH)
      def _():
        gath(j0 + 2, 0)

      @pl.when(j1 < NCH)
      def _():
        scat(j1, 1)

      return carry

    lax.fori_loop(0, (NCH + 1) // 2, body, 0)
    plsc.subcore_barrier()
    pltpu.sync_copy(acc.at[pl.ds(base, RPS)], out_hbm.at[c, pl.ds(base, RPS)])
    if with_deg:
      pltpu.sync_copy(accd.at[pl.ds(base, RPS)],
                      outd_hbm.at[c, pl.ds(base, RPS)])

  return agg


_AGG1 = _make_agg(True, jnp.bfloat16)
_AGG2 = _make_agg(False, jnp.bfloat16)

_R = 1000


def _tc1(x, parts1, degp, w1s, w1n, b1, w2s, w2n, b2):
  def body(x_ref, p_ref, d_ref, w1s_ref, w1n_ref, b1_ref, w2s_ref, w2n_ref,
           b2_ref, p2_ref, s2_ref):
    agg = (p_ref[0].astype(jnp.float32)
           + p_ref[1].astype(jnp.float32))
    deg = d_ref[0, :, :1] + d_ref[1, :, :1]
    hn = agg * (1.0 / jnp.maximum(deg, 1.0))
    h1 = jnp.maximum(
        jnp.dot(x_ref[...], w1s_ref[...], preferred_element_type=jnp.float32)
        + jnp.dot(hn, w1n_ref[...], preferred_element_type=jnp.float32)
        + b1_ref[...], 0.0)
    p2_ref[...] = jnp.dot(
        h1, w2n_ref[...],
        preferred_element_type=jnp.float32).astype(jnp.bfloat16)
    s2_ref[...] = (jnp.dot(h1, w2s_ref[...], preferred_element_type=jnp.float32)
                   + b2_ref[...])

  return pl.pallas_call(
      body,
      grid=(N // _R,),
      in_specs=[
          pl.BlockSpec((_R, D_IN), lambda i: (i, 0)),
          pl.BlockSpec((NC, _R, D_IN), lambda i: (0, i, 0)),
          pl.BlockSpec((NC, _R, DD), lambda i: (0, i, 0)),
          pl.BlockSpec((D_IN, D_HID), lambda i: (0, 0)),
          pl.BlockSpec((D_IN, D_HID), lambda i: (0, 0)),
          pl.BlockSpec((1, D_HID), lambda i: (0, 0)),
          pl.BlockSpec((D_HID, D_OUT), lambda i: (0, 0)),
          pl.BlockSpec((D_HID, D_OUT), lambda i: (0, 0)),
          pl.BlockSpec((1, D_OUT), lambda i: (0, 0)),
      ],
      out_specs=[
          pl.BlockSpec((_R, D_OUT), lambda i: (i, 0)),
          pl.BlockSpec((_R, D_OUT), lambda i: (i, 0)),
      ],
      out_shape=[
          jax.ShapeDtypeStruct((N, D_OUT), jnp.bfloat16),
          jax.ShapeDtypeStruct((N, D_OUT), jnp.float32),
      ],
  )(x, parts1, degp, w1s, w1n, b1, w2s, w2n, b2)


def _tc2(s2, parts2, degp):
  def body(s2_ref, p_ref, d_ref, o_ref):
    deg = d_ref[0, :, :1] + d_ref[1, :, :1]
    agg = p_ref[0].astype(jnp.float32) + p_ref[1].astype(jnp.float32)
    o_ref[...] = s2_ref[...] + agg * (1.0 / jnp.maximum(deg, 1.0))

  return pl.pallas_call(
      body,
      grid=(N // _R,),
      in_specs=[
          pl.BlockSpec((_R, D_OUT), lambda i: (i, 0)),
          pl.BlockSpec((NC, _R, D_OUT), lambda i: (0, i, 0)),
          pl.BlockSpec((NC, _R, DD), lambda i: (0, i, 0)),
      ],
      out_specs=pl.BlockSpec((_R, D_OUT), lambda i: (i, 0)),
      out_shape=jax.ShapeDtypeStruct((N, D_OUT), jnp.float32),
  )(s2, parts2, degp)


def kernel(in_feat, edge_index, W1_self, W1_neigh, b1, W2_self, W2_neigh, b2):
  pad = ((0, 0), (0, EWP - EW))
  src = jnp.pad(edge_index[0].astype(jnp.int32).reshape(NW, EW), pad,
                constant_values=0).reshape(NW, NCH, K)
  dst = jnp.pad(edge_index[1].astype(jnp.int32).reshape(NW, EW), pad,
                constant_values=N).reshape(NW, NCH, K)
  zeros128 = jnp.zeros((N, D_IN), jnp.bfloat16)
  zeros8 = jnp.zeros((N, DD), jnp.float32)
  ones = jnp.ones((K, DD), jnp.float32)
  parts1, degp = _AGG1(in_feat.astype(jnp.bfloat16), src, dst, zeros128,
                       zeros8, ones)
  p2, s2 = _tc1(in_feat, parts1, degp, W1_self.T, W1_neigh.T,
                b1.reshape(1, -1), W2_self.T, W2_neigh.T, b2.reshape(1, -1))
  (parts2,) = _AGG2(p2, src, dst, zeros128)
  return _tc2(s2, parts2, degp)
